# Initial kernel scaffold; baseline (speedup 1.0000x reference)
import jax
import jax.numpy as jnp
from jax.experimental import pallas as pl


def kernel(input):
    raise NotImplementedError("write your pallas kernel here")


# speedup vs baseline: 2.0974x; 2.0974x over previous
import jax
import jax.numpy as jnp
from jax.experimental import pallas as pl

NC = 384
CB = 128
HW = 56 * 56


def _rev_body(x_ref, o_ref):
    r = jax.lax.broadcasted_iota(jnp.int32, (CB, CB), 0)
    c = jax.lax.broadcasted_iota(jnp.int32, (CB, CB), 1)
    p = (r + c == CB - 1).astype(jnp.float32)
    o_ref[...] = jax.lax.dot(
        p, x_ref[0], preferred_element_type=jnp.float32
    )[None]


def kernel(input):
    b, c, h, w = input.shape
    x = input.reshape(b, c, h * w)
    nblk = c // CB
    out = pl.pallas_call(
        _rev_body,
        grid=(b, nblk),
        in_specs=[pl.BlockSpec((1, CB, h * w), lambda i, j: (i, nblk - 1 - j, 0))],
        out_specs=pl.BlockSpec((1, CB, h * w), lambda i, j: (i, j, 0)),
        out_shape=jax.ShapeDtypeStruct((b, c, h * w), input.dtype),
    )(x)
    return out.reshape(b, c, h, w)
